# baseline (speedup 1.0000x reference)
import jax
import jax.numpy as jnp
from jax.experimental import pallas as pl
from jax.experimental.pallas import tpu as pltpu

_DEPTH = 1000
_LANES = 1024
_SUB = 1000
_ROWS = 1024
_CMAX = 24


def _one_hot_flat_block(idx_ref, val_ref, out_ref):
    idx_blk = idx_ref[...]
    s = jax.lax.broadcasted_iota(jnp.int32, (_SUB, 1), 0)
    x = s * 24
    c0 = jax.lax.shift_right_logical(x * 8389, 23)
    idxA = jnp.zeros((_SUB, 1), jnp.int32)
    idxB = jnp.zeros((_SUB, 1), jnp.int32)
    idxC = jnp.zeros((_SUB, 1), jnp.int32)
    for c in range(_CMAX):
        m = c0 == c
        slc_a = jax.lax.slice(idx_blk, (c, 0), (c + _SUB, 1))
        slc_b = jax.lax.slice(idx_blk, (c + 1, 0), (c + 1 + _SUB, 1))
        idxA = jnp.where(m, slc_a, idxA)
        idxB = jnp.where(m, slc_b, idxB)
        if c <= _CMAX - 2:
            slc_c = jax.lax.slice(idx_blk, (c + 2, 0), (c + 2 + _SUB, 1))
            idxC = jnp.where(m, slc_c, idxC)
    lstar = 1000 * (c0 + 1) - x
    t_a = idxA + lstar - 1000
    t_b = idxB + lstar
    t_c = idxC + lstar + 1000
    lane = jax.lax.broadcasted_iota(jnp.int32, (_SUB, _LANES), 1)
    t = jnp.where(lane < lstar, t_a, jnp.where(lane < lstar + 1000, t_b, t_c))
    off = val_ref[0]
    on = val_ref[1]
    out_ref[...] = jnp.where(lane == t, on, off)


def kernel(indices, values):
    n = indices.size
    n_sub = n * _DEPTH // _LANES
    grid = n // _ROWS
    idx_flat = indices.reshape(n, 1)
    out = pl.pallas_call(
        _one_hot_flat_block,
        grid=(grid,),
        in_specs=[
            pl.BlockSpec((_ROWS, 1), lambda i: (i, 0)),
            pl.BlockSpec(memory_space=pltpu.SMEM),
        ],
        out_specs=pl.BlockSpec((_SUB, _LANES), lambda i: (i, 0)),
        out_shape=jax.ShapeDtypeStruct((n_sub, _LANES), jnp.float32),
    )(idx_flat, values)
    return out.reshape(*indices.shape, _DEPTH)


# speedup vs baseline: 1.4338x; 1.4338x over previous
import functools

import jax
import jax.numpy as jnp
from jax import lax
from jax.experimental import pallas as pl
from jax.experimental.pallas import tpu as pltpu
from jax.experimental.pallas import tpu_sc as plsc

_DEPTH = 1000
_N = 26624
_NW = 32
_RPW = _N // _NW
_RBLK = 32
_NBLK = _RPW // _RBLK
_BUFW = _RBLK * _DEPTH


def _sc_body(idx_hbm, off_hbm, on_hbm, out_hbm, idx_v, off_v, on_v,
             buf0, buf1, sem0, sem1, sem_in):
    wid = lax.axis_index("s") * 2 + lax.axis_index("c")
    base_row = wid * _RPW

    pltpu.async_copy(idx_hbm.at[pl.ds(base_row, _RPW)], idx_v, sem_in).wait()
    pltpu.async_copy(off_hbm, off_v, sem_in).wait()
    pltpu.async_copy(on_hbm, on_v, sem_in).wait()

    off16 = off_v[...]
    on16 = on_v[...]
    lane = lax.iota(jnp.int32, 16)
    rowoff = lane * _DEPTH

    bufs = (buf0, buf1)
    sems = (sem0, sem1)

    def fill(i, _):
        buf0[pl.ds(i * 16, 16)] = off16
        buf1[pl.ds(i * 16, 16)] = off16
        return 0
    lax.fori_loop(0, _BUFW // 16, fill, 0)

    ngrp = _RBLK // 16

    def positions(b, g):
        idx16 = idx_v[pl.ds(b * _RBLK + g * 16, 16)]
        return rowoff + g * 16 * _DEPTH + idx16

    copies = [None, None]
    for b in range(_NBLK):
        p = b % 2
        buf = bufs[p]
        if copies[p] is not None:
            copies[p].wait()
            for g in range(ngrp):
                plsc.store_scatter(buf, [positions(b - 2, g)], off16)
        for g in range(ngrp):
            plsc.store_scatter(buf, [positions(b, g)], on16)
        dst = out_hbm.at[pl.ds((base_row + b * _RBLK) * _DEPTH, _BUFW)]
        copies[p] = pltpu.async_copy(buf, dst, sems[p])
    copies[_NBLK % 2].wait()
    copies[(_NBLK + 1) % 2].wait()


def kernel(indices, values):
    idx_flat = indices.reshape(-1)
    off16 = jnp.full((16,), values[0], jnp.float32)
    on16 = jnp.full((16,), values[1], jnp.float32)
    mesh = plsc.VectorSubcoreMesh(core_axis_name="c", subcore_axis_name="s")
    f = pl.kernel(
        _sc_body,
        out_type=jax.ShapeDtypeStruct((_N * _DEPTH,), jnp.float32),
        mesh=mesh,
        compiler_params=pltpu.CompilerParams(needs_layout_passes=False),
        scratch_types=[
            pltpu.VMEM((_RPW,), jnp.int32),
            pltpu.VMEM((16,), jnp.float32),
            pltpu.VMEM((16,), jnp.float32),
            pltpu.VMEM((_BUFW,), jnp.float32),
            pltpu.VMEM((_BUFW,), jnp.float32),
            pltpu.SemaphoreType.DMA,
            pltpu.SemaphoreType.DMA,
            pltpu.SemaphoreType.DMA,
        ],
    )
    out = f(idx_flat, off16, on16)
    return out.reshape(*indices.shape, _DEPTH)
